# baseline (speedup 1.0000x reference)
import functools

import jax
import jax.numpy as jnp
from jax import lax
from jax.experimental import pallas as pl
from jax.experimental.pallas import tpu as pltpu
from jax.experimental.pallas import tpu_sc as plsc


def _gather_kernel(B, H, D, num_workers, cb):
    bat_w = B // num_workers
    n_chunks = bat_w // cb
    mesh = plsc.VectorSubcoreMesh(core_axis_name="c", subcore_axis_name="s")

    @functools.partial(
        pl.kernel,
        mesh=mesh,
        out_type=jax.ShapeDtypeStruct((B, H, D), jnp.float32),
        scratch_types=[
            pltpu.VMEM((bat_w, H), jnp.int32),
            pltpu.VMEM((cb, H, D), jnp.float32),
            pltpu.SemaphoreType.DMA,
        ],
        compiler_params=pltpu.CompilerParams(use_tc_tiling_on_sc=False),
    )
    def k(idx_hbm, table_hbm, out_hbm, idx_v, rows_v, sem):
        nc = lax.axis_size("c")
        wid = lax.axis_index("s") * nc + lax.axis_index("c")
        bbase = wid * bat_w
        pltpu.sync_copy(idx_hbm.at[pl.ds(bbase, bat_w)], idx_v)

        def body(c, carry):
            for j in range(cb):
                pltpu.async_copy(
                    table_hbm.at[idx_v.at[c * cb + j]], rows_v.at[j], sem
                )
            for j in range(cb):
                pltpu.make_async_copy(
                    table_hbm.at[idx_v.at[0]], rows_v.at[j], sem
                ).wait()
            pltpu.sync_copy(rows_v, out_hbm.at[pl.ds(bbase + c * cb, cb)])
            return carry

        lax.fori_loop(0, n_chunks, body, 0)

    return k


def _detile_kernel(V, D, W):
    n_blk = (V + W - 1) // W

    def body(wt_ref, out_ref):
        t = jnp.transpose(wt_ref[...], (1, 0))
        t4 = t.reshape(W // 4, 4, D)
        out_ref[...] = jnp.concatenate([t4[:, k, :] for k in range(4)], axis=1)

    return pl.pallas_call(
        body,
        grid=(n_blk,),
        in_specs=[pl.BlockSpec((D, W), lambda i: (0, i))],
        out_specs=pl.BlockSpec((W // 4, 128), lambda i: (i, 0)),
        out_shape=jax.ShapeDtypeStruct((V * D // 128, 128), jnp.float32),
    )


def kernel(indexs, weights):
    B, H = indexs.shape
    V, D = weights.shape
    table = _detile_kernel(V, D, 1024)(weights.T).reshape(V, D)
    out = _gather_kernel(B, H, D, 32, 16)(indexs.astype(jnp.int32), table)
    return out


# speedup vs baseline: 1.2448x; 1.2448x over previous
import functools

import jax
import jax.numpy as jnp
from jax import lax
from jax.experimental import pallas as pl
from jax.experimental.pallas import tpu as pltpu
from jax.experimental.pallas import tpu_sc as plsc


def _gather_kernel(B, H, D, num_workers, cb):
    bat_w = B // num_workers
    n_chunks = bat_w // cb
    mesh = plsc.VectorSubcoreMesh(core_axis_name="c", subcore_axis_name="s")

    @functools.partial(
        pl.kernel,
        mesh=mesh,
        out_type=jax.ShapeDtypeStruct((B, H, D), jnp.float32),
        scratch_types=[
            pltpu.VMEM((bat_w, H), jnp.int32),
            pltpu.VMEM((cb, H, D), jnp.float32),
            pltpu.SemaphoreType.DMA,
        ],
        compiler_params=pltpu.CompilerParams(use_tc_tiling_on_sc=False),
    )
    def k(idx_hbm, table_hbm, out_hbm, idx_v, rows_v, sem):
        nc = lax.axis_size("c")
        wid = lax.axis_index("s") * nc + lax.axis_index("c")
        bbase = wid * bat_w
        pltpu.sync_copy(idx_hbm.at[pl.ds(bbase, bat_w)], idx_v)

        def body(c, carry):
            for j in range(cb):
                pltpu.async_copy(
                    table_hbm.at[idx_v.at[c * cb + j]], rows_v.at[j], sem
                )
            for j in range(cb):
                pltpu.make_async_copy(
                    table_hbm.at[idx_v.at[0]], rows_v.at[j], sem
                ).wait()
            pltpu.sync_copy(rows_v, out_hbm.at[pl.ds(bbase + c * cb, cb)])
            return carry

        lax.fori_loop(0, n_chunks, body, 0)

    return k


def kernel(indexs, weights):
    B, H = indexs.shape
    V, D = weights.shape
    idx = indexs.astype(jnp.int32)
    half = B // 2
    gk = _gather_kernel(half, H, D, 32, 16)
    out0 = gk(idx[:half], weights)
    out1 = gk(idx[half:], weights)
    return jnp.concatenate([out0, out1], axis=0)


# speedup vs baseline: 1.2893x; 1.0357x over previous
import functools

import jax
import jax.numpy as jnp
from jax import lax
from jax.experimental import pallas as pl
from jax.experimental.pallas import tpu as pltpu
from jax.experimental.pallas import tpu_sc as plsc


def _gather_kernel(B, H, D, num_workers, cb):
    bat_w = B // num_workers
    n_chunks = bat_w // cb
    mesh = plsc.VectorSubcoreMesh(core_axis_name="c", subcore_axis_name="s")

    @functools.partial(
        pl.kernel,
        mesh=mesh,
        out_type=jax.ShapeDtypeStruct((B, H, D), jnp.float32),
        scratch_types=[
            pltpu.VMEM((bat_w, H), jnp.int32),
            pltpu.VMEM((cb, H, D), jnp.float32),
            pltpu.SemaphoreType.DMA,
        ],
        compiler_params=pltpu.CompilerParams(use_tc_tiling_on_sc=False),
    )
    def k(idx_hbm, table_hbm, out_hbm, idx_v, rows_v, sem):
        nc = lax.axis_size("c")
        wid = lax.axis_index("s") * nc + lax.axis_index("c")
        bbase = wid * bat_w
        pltpu.sync_copy(idx_hbm.at[pl.ds(bbase, bat_w)], idx_v)

        def body(c, carry):
            for j in range(cb):
                pltpu.async_copy(
                    table_hbm.at[idx_v.at[c * cb + j]], rows_v.at[j], sem
                )
            for j in range(cb):
                pltpu.make_async_copy(
                    table_hbm.at[idx_v.at[0]], rows_v.at[j], sem
                ).wait()
            pltpu.sync_copy(rows_v, out_hbm.at[pl.ds(bbase + c * cb, cb)])
            return carry

        lax.fori_loop(0, n_chunks, body, 0)

    return k


def kernel(indexs, weights):
    B, H = indexs.shape
    V, D = weights.shape
    idx = indexs.astype(jnp.int32)
    nsplit = 4
    part = B // nsplit
    gk = _gather_kernel(part, H, D, 32, 16)
    outs = [gk(idx[i * part : (i + 1) * part], weights) for i in range(nsplit)]
    return jnp.concatenate(outs, axis=0)
